# baseline (speedup 1.0000x reference)
import jax
import jax.numpy as jnp
from jax import lax
from jax.experimental import pallas as pl
from jax.experimental.pallas import tpu as pltpu
from jax.experimental.pallas import tpu_sc as plsc

_B, _T = 4096, 50
_D = 64
_G = 128
_NB = 5
_LA = 3

_info = plsc.get_sparse_core_info()
_NC, _NS = _info.num_cores, _info.num_subcores
_NW = _NC * _NS
_N = _B * _T * 2
_NG = _N // _G
_GPW = _NG // _NW


def _gather_body(table_hbm, idx_hbm, out_hbm, idx_v, rows_v, *sems):
    gsem, wsem = sems[:_NB], sems[_NB:]
    wid = lax.axis_index("s") * _NC + lax.axis_index("c")
    gbase = wid * _GPW
    pltpu.sync_copy(idx_hbm.at[pl.ds(gbase, _GPW)], idx_v)

    def gfire(j, b):
        pltpu.async_copy(table_hbm.at[idx_v.at[j]], rows_v.at[b], gsem[b])

    def gwait(b):
        pltpu.make_async_copy(
            table_hbm.at[idx_v.at[0]], rows_v.at[b], gsem[b]).wait()

    def wfire(j, b):
        h = gbase + j
        t = h // (2 * _B // _G)
        r = h % (2 * _B // _G)
        row0 = t * _B + (r // 2) * _G
        col0 = (r % 2) * _D
        pltpu.async_copy(
            rows_v.at[b],
            out_hbm.at[pl.ds(row0, _G), pl.ds(col0, _D)],
            wsem[b])

    def wwait(b):
        pltpu.make_async_copy(
            rows_v.at[b],
            out_hbm.at[pl.ds(0, _G), pl.ds(0, _D)],
            wsem[b]).wait()

    for j in range(_LA):
        gfire(j, j % _NB)

    def step(j, b, fire_next, wait_wb):
        gwait(b)
        wfire(j, b)
        if fire_next:
            bb = (b + _LA) % _NB
            if wait_wb:
                wwait(bb)
            gfire(j + _LA, bb)

    for b in range(_NB):
        step(b, b, True, b >= _NB - _LA)

    def body(i, carry):
        for b in range(_NB):
            step(_NB * i + b, b, True, True)
        return carry

    lax.fori_loop(1, _GPW // _NB - 1, body, 0)

    for b in range(_NB):
        j = _GPW - _NB + b
        step(j, b, j + _LA < _GPW, True)

    for b in range(_NB):
        wwait(b)


_V = 100000
_TW = 16384


def _tr_body(x_ref, o_ref):
    b3 = x_ref[...].T.reshape(_TW // 2, 2, _D)
    o_ref[...] = jnp.concatenate([b3[:, p, :] for p in range(2)], axis=1)


def _transpose_tc(wnat):
    return pl.pallas_call(
        _tr_body,
        grid=(pl.cdiv(_V, _TW),),
        in_specs=[pl.BlockSpec((_D, _TW), lambda g: (0, g))],
        out_specs=pl.BlockSpec((_TW // 2, 2 * _D), lambda g: (g, 0)),
        out_shape=jax.ShapeDtypeStruct((_V // 2, 2 * _D), jnp.float32),
    )(wnat)


@jax.jit
def _dual_embed(table, idx):
    run = pl.kernel(
        _gather_body,
        out_type=jax.ShapeDtypeStruct((_T * _B, 2 * _D), jnp.float32),
        mesh=plsc.VectorSubcoreMesh(core_axis_name="c", subcore_axis_name="s"),
        scratch_types=[
            pltpu.VMEM((_GPW, _G), jnp.int32),
            pltpu.VMEM((_NB, _G, _D), jnp.float32),
        ] + [pltpu.SemaphoreType.DMA] * (2 * _NB),
        compiler_params=pltpu.CompilerParams(
            use_tc_tiling_on_sc=False, needs_layout_passes=False),
    )
    return run(table, idx)


def kernel(token, embedding_weight):
    idx = (token.astype(jnp.int32)
           .reshape(32, 128, _T, 2).transpose(2, 0, 3, 1).reshape(_NG, _G))
    table_lin = _transpose_tc(embedding_weight.T).reshape(_V, _D)
    out = _dual_embed(table_lin, idx)
    return out.reshape(_T, _B, 2 * _D).transpose(1, 0, 2)


# speedup vs baseline: 1.0225x; 1.0225x over previous
import jax
import jax.numpy as jnp
from jax import lax
from jax.experimental import pallas as pl
from jax.experimental.pallas import tpu as pltpu
from jax.experimental.pallas import tpu_sc as plsc

_B, _T = 4096, 50
_D = 64
_G = 128
_NB = 5
_LA = 3

_info = plsc.get_sparse_core_info()
_NC, _NS = _info.num_cores, _info.num_subcores
_NW = _NC * _NS
_N = _B * _T * 2
_NG = _N // _G
_GPW = _NG // _NW


def _gather_body(table_hbm, idx_hbm, out_hbm, idx_v, rows_v, *sems):
    gsem, wsem = sems[:_NB], sems[_NB:]
    wid = lax.axis_index("s") * _NC + lax.axis_index("c")
    gbase = wid * _GPW
    pltpu.sync_copy(idx_hbm.at[pl.ds(gbase, _GPW)], idx_v)

    def gfire(j, b):
        pltpu.async_copy(table_hbm.at[idx_v.at[j]], rows_v.at[b], gsem[b])

    def gwait(b):
        pltpu.make_async_copy(
            table_hbm.at[idx_v.at[0]], rows_v.at[b], gsem[b]).wait()

    def wfire(j, b):
        h = gbase + j
        t = h // (2 * _B // _G)
        r = h % (2 * _B // _G)
        row0 = t * _B + (r // 2) * _G
        col0 = (r % 2) * _D
        pltpu.async_copy(
            rows_v.at[b],
            out_hbm.at[pl.ds(row0, _G), pl.ds(col0, _D)],
            wsem[b])

    def wwait(b):
        pltpu.make_async_copy(
            rows_v.at[b],
            out_hbm.at[pl.ds(0, _G), pl.ds(0, _D)],
            wsem[b]).wait()

    for j in range(_LA):
        gfire(j, j % _NB)

    def step(j, b, fire_next, wait_wb):
        gwait(b)
        wfire(j, b)
        if fire_next:
            bb = (b + _LA) % _NB
            if wait_wb:
                wwait(bb)
            gfire(j + _LA, bb)

    for b in range(_NB):
        step(b, b, True, b >= _NB - _LA)

    def body(i, carry):
        for b in range(_NB):
            step(_NB * i + b, b, True, True)
        return carry

    lax.fori_loop(1, _GPW // _NB - 1, body, 0)

    for b in range(_NB):
        j = _GPW - _NB + b
        step(j, b, j + _LA < _GPW, True)

    for b in range(_NB):
        wwait(b)


_V = 100000
_TW = 4096


def _tr_body(x_ref, o_ref):
    b3 = x_ref[...].T.reshape(_TW // 2, 2, _D)
    o_ref[...] = jnp.concatenate([b3[:, p, :] for p in range(2)], axis=1)


def _transpose_tc(wnat):
    return pl.pallas_call(
        _tr_body,
        grid=(pl.cdiv(_V, _TW),),
        in_specs=[pl.BlockSpec((_D, _TW), lambda g: (0, g))],
        out_specs=pl.BlockSpec((_TW // 2, 2 * _D), lambda g: (g, 0)),
        out_shape=jax.ShapeDtypeStruct((_V // 2, 2 * _D), jnp.float32),
    )(wnat)


@jax.jit
def _dual_embed(table, idx):
    run = pl.kernel(
        _gather_body,
        out_type=jax.ShapeDtypeStruct((_T * _B, 2 * _D), jnp.float32),
        mesh=plsc.VectorSubcoreMesh(core_axis_name="c", subcore_axis_name="s"),
        scratch_types=[
            pltpu.VMEM((_GPW, _G), jnp.int32),
            pltpu.VMEM((_NB, _G, _D), jnp.float32),
        ] + [pltpu.SemaphoreType.DMA] * (2 * _NB),
        compiler_params=pltpu.CompilerParams(
            use_tc_tiling_on_sc=False, needs_layout_passes=False),
    )
    return run(table, idx)


def kernel(token, embedding_weight):
    idx = (token.astype(jnp.int32)
           .reshape(32, 128, _T, 2).transpose(2, 0, 3, 1).reshape(_NG, _G))
    table_lin = _transpose_tc(embedding_weight.T).reshape(_V, _D)
    out = _dual_embed(table_lin, idx)
    return out.reshape(_T, _B, 2 * _D).transpose(1, 0, 2)


# speedup vs baseline: 1.0233x; 1.0007x over previous
import jax
import jax.numpy as jnp
from jax import lax
from jax.experimental import pallas as pl
from jax.experimental.pallas import tpu as pltpu
from jax.experimental.pallas import tpu_sc as plsc

_B, _T = 4096, 50
_D = 64
_G = 128
_NB = 5
_LA = 3

_info = plsc.get_sparse_core_info()
_NC, _NS = _info.num_cores, _info.num_subcores
_NW = _NC * _NS
_N = _B * _T * 2
_NG = _N // _G
_GPW = _NG // _NW


def _gather_body(table_hbm, idx_hbm, out_hbm, idx_v, rows_v, *sems):
    gsem, wsem = sems[:_NB], sems[_NB:]
    wid = lax.axis_index("s") * _NC + lax.axis_index("c")
    gbase = wid * _GPW
    pltpu.sync_copy(idx_hbm.at[pl.ds(gbase, _GPW)], idx_v)

    def gfire(j, b):
        pltpu.async_copy(table_hbm.at[idx_v.at[j]], rows_v.at[b], gsem[b])

    def gwait(b):
        pltpu.make_async_copy(
            table_hbm.at[idx_v.at[0]], rows_v.at[b], gsem[b]).wait()

    def wfire(j, b):
        h = gbase + j
        t = h // (2 * _B // _G)
        r = h % (2 * _B // _G)
        row0 = t * _B + (r // 2) * _G
        col0 = (r % 2) * _D
        pltpu.async_copy(
            rows_v.at[b],
            out_hbm.at[pl.ds(row0, _G), pl.ds(col0, _D)],
            wsem[b])

    def wwait(b):
        pltpu.make_async_copy(
            rows_v.at[b],
            out_hbm.at[pl.ds(0, _G), pl.ds(0, _D)],
            wsem[b]).wait()

    for j in range(_LA):
        gfire(j, j % _NB)

    def step(j, b, fire_next, wait_wb):
        gwait(b)
        wfire(j, b)
        if fire_next:
            bb = (b + _LA) % _NB
            if wait_wb:
                wwait(bb)
            gfire(j + _LA, bb)

    for b in range(_NB):
        step(b, b, True, b >= _NB - _LA)

    def body(i, carry):
        for b in range(_NB):
            step(_NB * i + b, b, True, True)
        return carry

    lax.fori_loop(1, _GPW // _NB - 1, body, 0)

    for b in range(_NB):
        j = _GPW - _NB + b
        step(j, b, j + _LA < _GPW, True)

    for b in range(_NB):
        wwait(b)


_V = 100000
_TW = 8192


def _tr_body(x_ref, o_ref):
    b3 = x_ref[...].T.reshape(_TW // 2, 2, _D)
    o_ref[...] = jnp.concatenate([b3[:, p, :] for p in range(2)], axis=1)


def _transpose_tc(wnat):
    return pl.pallas_call(
        _tr_body,
        grid=(pl.cdiv(_V, _TW),),
        in_specs=[pl.BlockSpec((_D, _TW), lambda g: (0, g))],
        out_specs=pl.BlockSpec((_TW // 2, 2 * _D), lambda g: (g, 0)),
        out_shape=jax.ShapeDtypeStruct((_V // 2, 2 * _D), jnp.float32),
    )(wnat)


@jax.jit
def _dual_embed(table, idx):
    run = pl.kernel(
        _gather_body,
        out_type=jax.ShapeDtypeStruct((_T * _B, 2 * _D), jnp.float32),
        mesh=plsc.VectorSubcoreMesh(core_axis_name="c", subcore_axis_name="s"),
        scratch_types=[
            pltpu.VMEM((_GPW, _G), jnp.int32),
            pltpu.VMEM((_NB, _G, _D), jnp.float32),
        ] + [pltpu.SemaphoreType.DMA] * (2 * _NB),
        compiler_params=pltpu.CompilerParams(
            use_tc_tiling_on_sc=False, needs_layout_passes=False),
    )
    return run(table, idx)


def kernel(token, embedding_weight):
    idx = (token.astype(jnp.int32)
           .reshape(32, 128, _T, 2).transpose(2, 0, 3, 1).reshape(_NG, _G))
    table_lin = _transpose_tc(embedding_weight.T).reshape(_V, _D)
    out = _dual_embed(table_lin, idx)
    return out.reshape(_T, _B, 2 * _D).transpose(1, 0, 2)
